# Initial kernel scaffold; baseline (speedup 1.0000x reference)
import jax
import jax.numpy as jnp
from jax.experimental import pallas as pl


def kernel(indices, table):
    raise NotImplementedError("write your pallas kernel here")


# speedup vs baseline: 3.3258x; 3.3258x over previous
import functools

import jax
import jax.numpy as jnp
from jax import lax
from jax.experimental import pallas as pl
from jax.experimental.pallas import tpu as pltpu
from jax.experimental.pallas import tpu_sc as plsc

B, L, G = 1024, 200, 3
D = 64
N = B * L * G
LANES = 16
NW = 32
ROWS_PER_W = N // NW
C = 128
N_CHUNKS = ROWS_PER_W // C


def _relu_body(t_ref, o_ref):
    o_ref[...] = jnp.maximum(t_ref[...], 0.0)


def _relu_table(table):
    V = table.shape[0]
    BLK = 8192
    return pl.pallas_call(
        _relu_body,
        grid=(pl.cdiv(V, BLK),),
        in_specs=[pl.BlockSpec((BLK, D), lambda i: (i, 0))],
        out_specs=pl.BlockSpec((BLK, D), lambda i: (i, 0)),
        out_shape=jax.ShapeDtypeStruct((V, D), table.dtype),
    )(table)


_mesh = plsc.VectorSubcoreMesh(core_axis_name="c", subcore_axis_name="s")


@functools.partial(
    pl.kernel,
    out_type=jax.ShapeDtypeStruct((N, D), jnp.float32),
    mesh=_mesh,
    scratch_types=[
        pltpu.VMEM((C,), jnp.int32),
        pltpu.VMEM((C,), jnp.int32),
        pltpu.VMEM((C, D), jnp.float32),
        pltpu.SemaphoreType.DMA,
        pltpu.SemaphoreType.DMA,
    ],
    compiler_params=pltpu.CompilerParams(use_tc_tiling_on_sc=False),
)
def _gather_scatter(idx_hbm, table_hbm, out_hbm, idx_v, sidx_v, rows_v, gsem, ssem):
    cid = lax.axis_index("c")
    sid = lax.axis_index("s")
    wid = sid * 2 + cid
    wbase = wid * ROWS_PER_W
    b_start = wid * (ROWS_PER_W // (L * G))

    @pl.loop(0, N_CHUNKS, init_carry=(b_start, jnp.int32(0)))
    def _chunk(i, carry):
        b0, rem0 = carry
        base = pl.multiple_of(wbase + i * C, C)
        pltpu.sync_copy(idx_hbm.at[pl.ds(base, C)], idx_v)
        for v in range(C // LANES):
            off = lax.iota(jnp.int32, LANES) + (v * LANES)
            t = rem0 + off
            wrap = 1 + lax.shift_right_arithmetic(t - L * G, 31)
            b = b0 + wrap
            rem = t - wrap * (L * G)
            l = lax.shift_right_logical(rem * 21846, 16)
            g = rem - l * G
            sidx_v[pl.ds(v * LANES, LANES)] = l * (B * G) + b * G + g
        pltpu.async_copy(table_hbm.at[idx_v], rows_v, gsem).wait()
        pltpu.async_copy(rows_v, out_hbm.at[sidx_v], ssem).wait()
        rem1 = rem0 + (C % (L * G))
        wrap1 = 1 + lax.shift_right_arithmetic(rem1 - L * G, 31)
        return (b0 + wrap1, rem1 - wrap1 * (L * G))


def kernel(indices, table):
    rtable = _relu_table(table)
    idx_flat = indices.astype(jnp.int32).reshape(-1)
    out = _gather_scatter(idx_flat, rtable)
    return out.reshape(L, B, G * D)


# speedup vs baseline: 4.1854x; 1.2585x over previous
import functools

import jax
import jax.numpy as jnp
from jax import lax
from jax.experimental import pallas as pl
from jax.experimental.pallas import tpu as pltpu
from jax.experimental.pallas import tpu_sc as plsc

B, L, G = 1024, 200, 3
D = 64
N = B * L * G
LANES = 16
NW = 32
ROWS_PER_W = N // NW
C = 128
N_CHUNKS = ROWS_PER_W // C
NBUF = 6
K = 3


def _relu_body(t_ref, o_ref):
    o_ref[...] = jnp.maximum(t_ref[...], 0.0)


def _relu_table(table):
    V = table.shape[0]
    BLK = 8192
    return pl.pallas_call(
        _relu_body,
        grid=(pl.cdiv(V, BLK),),
        in_specs=[pl.BlockSpec((BLK, D), lambda i: (i, 0))],
        out_specs=pl.BlockSpec((BLK, D), lambda i: (i, 0)),
        out_shape=jax.ShapeDtypeStruct((V, D), table.dtype),
    )(table)


_mesh = plsc.VectorSubcoreMesh(core_axis_name="c", subcore_axis_name="s")

_scratch = (
    [pltpu.VMEM((C,), jnp.int32) for _ in range(NBUF)]
    + [pltpu.VMEM((C,), jnp.int32) for _ in range(NBUF)]
    + [pltpu.VMEM((C, D), jnp.float32) for _ in range(NBUF)]
    + [pltpu.SemaphoreType.DMA for _ in range(2 * NBUF)]
)


@functools.partial(
    pl.kernel,
    out_type=jax.ShapeDtypeStruct((N, D), jnp.float32),
    mesh=_mesh,
    scratch_types=_scratch,
    compiler_params=pltpu.CompilerParams(use_tc_tiling_on_sc=False),
)
def _gather_scatter(idx_hbm, table_hbm, out_hbm, *scratch):
    ibuf = scratch[0:NBUF]
    sidx = scratch[NBUF : 2 * NBUF]
    rows = scratch[2 * NBUF : 3 * NBUF]
    gsem = scratch[3 * NBUF : 4 * NBUF]
    ssem = scratch[4 * NBUF : 5 * NBUF]

    cid = lax.axis_index("c")
    sid = lax.axis_index("s")
    wid = sid * 2 + cid
    wbase = wid * ROWS_PER_W

    def compute_sidx(b, b0, rem0):
        for v in range(C // LANES):
            off = lax.iota(jnp.int32, LANES) + (v * LANES)
            t = rem0 + off
            wrap = 1 + lax.shift_right_arithmetic(t - L * G, 31)
            bb = b0 + wrap
            rem = t - wrap * (L * G)
            l = lax.shift_right_logical(rem * 21846, 16)
            g = rem - l * G
            sidx[b][pl.ds(v * LANES, LANES)] = l * (B * G) + bb * G + g
        rem1 = rem0 + (C % (L * G))
        wrap1 = 1 + lax.shift_right_arithmetic(rem1 - L * G, 31)
        return b0 + wrap1, rem1 - wrap1 * (L * G)

    def gather_start(c, b):
        base = pl.multiple_of(wbase + c * C, C)
        pltpu.sync_copy(idx_hbm.at[pl.ds(base, C)], ibuf[b])
        pltpu.async_copy(table_hbm.at[ibuf[b]], rows[b], gsem[b])

    def gather_wait(b):
        pltpu.make_async_copy(table_hbm.at[ibuf[b]], rows[b], gsem[b]).wait()

    def scatter_start(b):
        pltpu.async_copy(rows[b], out_hbm.at[sidx[b]], ssem[b])

    def scatter_wait(b):
        pltpu.make_async_copy(rows[b], out_hbm.at[sidx[b]], ssem[b]).wait()

    carry0 = (wid * (ROWS_PER_W // (L * G)), jnp.int32(0))

    b0, rem0 = carry0
    for c in range(NBUF):
        if c >= K:
            gather_wait(c - K)
            scatter_start(c - K)
        b0, rem0 = compute_sidx(c, b0, rem0)
        gather_start(c, c)

    @pl.loop(0, (N_CHUNKS - NBUF) // NBUF, init_carry=(b0, rem0))
    def _block(j, carry):
        b0, rem0 = carry
        c0 = NBUF + j * NBUF
        for b in range(NBUF):
            b2 = (b + NBUF - K) % NBUF
            gather_wait(b2)
            scatter_start(b2)
            scatter_wait(b)
            b0, rem0 = compute_sidx(b, b0, rem0)
            gather_start(c0 + b, b)
        return (b0, rem0)

    for c in range(N_CHUNKS - K, N_CHUNKS):
        b = c % NBUF
        gather_wait(b)
        scatter_start(b)
    for b in range(NBUF):
        scatter_wait(b)


def kernel(indices, table):
    rtable = _relu_table(table)
    idx_flat = indices.astype(jnp.int32).reshape(-1)
    out = _gather_scatter(idx_flat, rtable)
    return out.reshape(L, B, G * D)


# speedup vs baseline: 4.2423x; 1.0136x over previous
import functools

import jax
import jax.numpy as jnp
from jax import lax
from jax.experimental import pallas as pl
from jax.experimental.pallas import tpu as pltpu
from jax.experimental.pallas import tpu_sc as plsc

B, L, G = 1024, 200, 3
D = 64
N = B * L * G
LANES = 16
NW = 32
ROWS_PER_W = N // NW
C = 128
N_CHUNKS = ROWS_PER_W // C
NBUF = 6
K = 3


def _relu_body(t_ref, o_ref):
    o_ref[...] = jnp.maximum(t_ref[...], 0.0)


def _relu_table(table):
    V = table.shape[0]
    BLK = 8192
    return pl.pallas_call(
        _relu_body,
        grid=(pl.cdiv(V, BLK),),
        in_specs=[pl.BlockSpec((BLK, D), lambda i: (i, 0))],
        out_specs=pl.BlockSpec((BLK, D), lambda i: (i, 0)),
        out_shape=jax.ShapeDtypeStruct((V, D), table.dtype),
    )(table)


_mesh = plsc.VectorSubcoreMesh(core_axis_name="c", subcore_axis_name="s")

_scratch = (
    [pltpu.VMEM((C,), jnp.int32) for _ in range(NBUF)]
    + [pltpu.VMEM((C,), jnp.int32) for _ in range(NBUF)]
    + [pltpu.VMEM((C, D), jnp.float32) for _ in range(NBUF)]
    + [pltpu.SemaphoreType.DMA for _ in range(3 * NBUF)]
)


@functools.partial(
    pl.kernel,
    out_type=jax.ShapeDtypeStruct((N, D), jnp.float32),
    mesh=_mesh,
    scratch_types=_scratch,
    compiler_params=pltpu.CompilerParams(use_tc_tiling_on_sc=False),
)
def _gather_scatter(idx_hbm, table_hbm, out_hbm, *scratch):
    ibuf = scratch[0:NBUF]
    sidx = scratch[NBUF : 2 * NBUF]
    rows = scratch[2 * NBUF : 3 * NBUF]
    gsem = scratch[3 * NBUF : 4 * NBUF]
    ssem = scratch[4 * NBUF : 5 * NBUF]
    isem = scratch[5 * NBUF : 6 * NBUF]

    cid = lax.axis_index("c")
    sid = lax.axis_index("s")
    wid = sid * 2 + cid
    wbase = wid * ROWS_PER_W

    def compute_sidx(b, b0, rem0):
        for v in range(C // LANES):
            off = lax.iota(jnp.int32, LANES) + (v * LANES)
            t = rem0 + off
            wrap = 1 + lax.shift_right_arithmetic(t - L * G, 31)
            bb = b0 + wrap
            rem = t - wrap * (L * G)
            l = lax.shift_right_logical(rem * 21846, 16)
            g = rem - l * G
            sidx[b][pl.ds(v * LANES, LANES)] = l * (B * G) + bb * G + g
        rem1 = rem0 + (C % (L * G))
        wrap1 = 1 + lax.shift_right_arithmetic(rem1 - L * G, 31)
        return b0 + wrap1, rem1 - wrap1 * (L * G)

    def idx_start(c, b):
        base = pl.multiple_of(wbase + c * C, C)
        pltpu.async_copy(idx_hbm.at[pl.ds(base, C)], ibuf[b], isem[b])

    def idx_wait(b):
        pltpu.make_async_copy(idx_hbm.at[pl.ds(0, C)], ibuf[b], isem[b]).wait()

    def gather_start(b):
        pltpu.async_copy(table_hbm.at[ibuf[b]], rows[b], gsem[b])

    def gather_wait(b):
        pltpu.make_async_copy(table_hbm.at[ibuf[b]], rows[b], gsem[b]).wait()

    def scatter_start(b):
        pltpu.async_copy(rows[b], out_hbm.at[sidx[b]], ssem[b])

    def scatter_wait(b):
        pltpu.make_async_copy(rows[b], out_hbm.at[sidx[b]], ssem[b]).wait()

    carry0 = (wid * (ROWS_PER_W // (L * G)), jnp.int32(0))

    b0, rem0 = carry0
    for c in range(NBUF):
        idx_start(c, c)
    for c in range(NBUF):
        b = c
        if c >= K:
            b2 = c - K
            gather_wait(b2)
            scatter_start(b2)
            idx_start(c + K, b2)
        b0, rem0 = compute_sidx(b, b0, rem0)
        idx_wait(b)
        gather_start(b)

    @pl.loop(0, (N_CHUNKS - 2 * NBUF) // NBUF, init_carry=(b0, rem0))
    def _block(j, carry):
        b0, rem0 = carry
        for b in range(NBUF):
            b2 = (b + NBUF - K) % NBUF
            gather_wait(b2)
            scatter_start(b2)
            idx_start(NBUF + j * NBUF + b + K, b2)
            scatter_wait(b)
            b0, rem0 = compute_sidx(b, b0, rem0)
            idx_wait(b)
            gather_start(b)
        return (b0, rem0)

    bb0, brem0 = _block
    for c in range(N_CHUNKS - NBUF, N_CHUNKS):
        b = c % NBUF
        b2 = (b + NBUF - K) % NBUF
        gather_wait(b2)
        scatter_start(b2)
        if c + K < N_CHUNKS:
            idx_start(c + K, b2)
        scatter_wait(b)
        bb0, brem0 = compute_sidx(b, bb0, brem0)
        idx_wait(b)
        gather_start(b)

    for c in range(N_CHUNKS - K, N_CHUNKS):
        b = c % NBUF
        gather_wait(b)
        scatter_start(b)
    for b in range(NBUF):
        scatter_wait(b)


def kernel(indices, table):
    rtable = _relu_table(table)
    idx_flat = indices.astype(jnp.int32).reshape(-1)
    out = _gather_scatter(idx_flat, rtable)
    return out.reshape(L, B, G * D)


# speedup vs baseline: 5.1707x; 1.2189x over previous
import functools

import jax
import jax.numpy as jnp
from jax import lax
from jax.experimental import pallas as pl
from jax.experimental.pallas import tpu as pltpu
from jax.experimental.pallas import tpu_sc as plsc

B, L, G = 1024, 200, 3
D = 64
N = B * L * G
LANES = 16
NW = 32
ROWS_PER_W = N // NW
C = 128
N_CHUNKS = ROWS_PER_W // C
NBUF = 6
K = 3


def _relu_body(t_ref, o_ref):
    o_ref[...] = jnp.maximum(t_ref[...], 0.0)


def _relu_table(table):
    V = table.shape[0]
    BLK = 8192
    return pl.pallas_call(
        _relu_body,
        grid=(pl.cdiv(V, BLK),),
        in_specs=[pl.BlockSpec((BLK, D), lambda i: (i, 0))],
        out_specs=pl.BlockSpec((BLK, D), lambda i: (i, 0)),
        out_shape=jax.ShapeDtypeStruct((V, D), table.dtype),
    )(table)


_mesh = plsc.VectorSubcoreMesh(core_axis_name="c", subcore_axis_name="s")

_scratch = (
    [pltpu.VMEM((C,), jnp.int32) for _ in range(NBUF)]
    + [pltpu.VMEM((C,), jnp.int32) for _ in range(NBUF)]
    + [pltpu.VMEM((C, D), jnp.float32) for _ in range(NBUF)]
    + [pltpu.SemaphoreType.DMA for _ in range(3 * NBUF)]
)


@functools.partial(
    pl.kernel,
    out_type=jax.ShapeDtypeStruct((N, D), jnp.float32),
    mesh=_mesh,
    scratch_types=_scratch,
    compiler_params=pltpu.CompilerParams(use_tc_tiling_on_sc=False),
)
def _gather_scatter(idx_hbm, table_hbm, out_hbm, *scratch):
    ibuf = scratch[0:NBUF]
    sidx = scratch[NBUF : 2 * NBUF]
    rows = scratch[2 * NBUF : 3 * NBUF]
    gsem = scratch[3 * NBUF : 4 * NBUF]
    ssem = scratch[4 * NBUF : 5 * NBUF]
    isem = scratch[5 * NBUF : 6 * NBUF]

    cid = lax.axis_index("c")
    sid = lax.axis_index("s")
    wid = sid * 2 + cid
    wbase = wid * ROWS_PER_W

    def compute_sidx(b, cg):
        c8 = lax.shift_right_logical(cg, 3)
        g = lax.shift_right_logical(c8 * 328, 16)
        l = c8 - L * g
        b0 = lax.shift_left(cg & 7, 7)
        base_s = l * (B * G) + b0 * G + g
        for v in range(C // LANES):
            cv = (lax.iota(jnp.int32, LANES) + (v * LANES)) * G
            sidx[b][pl.ds(v * LANES, LANES)] = base_s + cv

    def idx_start(c, b):
        base = pl.multiple_of(wbase + c * C, C)
        pltpu.async_copy(idx_hbm.at[pl.ds(base, C)], ibuf[b], isem[b])

    def idx_wait(b):
        pltpu.make_async_copy(idx_hbm.at[pl.ds(0, C)], ibuf[b], isem[b]).wait()

    def gather_start(b):
        pltpu.async_copy(table_hbm.at[ibuf[b]], rows[b], gsem[b])

    def gather_wait(b):
        pltpu.make_async_copy(table_hbm.at[ibuf[b]], rows[b], gsem[b]).wait()

    def scatter_start(b):
        pltpu.async_copy(rows[b], out_hbm.at[sidx[b]], ssem[b])

    def scatter_wait(b):
        pltpu.make_async_copy(rows[b], out_hbm.at[sidx[b]], ssem[b]).wait()

    wchunk = wid * N_CHUNKS

    for c in range(NBUF):
        idx_start(c, c)
    for c in range(NBUF):
        b = c
        if c >= K:
            b2 = c - K
            gather_wait(b2)
            scatter_start(b2)
            idx_start(c + K, b2)
        compute_sidx(b, wchunk + c)
        idx_wait(b)
        gather_start(b)

    @pl.loop(0, (N_CHUNKS - 2 * NBUF) // NBUF)
    def _block(j):
        for b in range(NBUF):
            c = NBUF + j * NBUF + b
            b2 = (b + NBUF - K) % NBUF
            gather_wait(b2)
            scatter_start(b2)
            idx_start(c + K, b2)
            scatter_wait(b)
            compute_sidx(b, wchunk + c)
            idx_wait(b)
            gather_start(b)

    for c in range(N_CHUNKS - NBUF, N_CHUNKS):
        b = c % NBUF
        b2 = (b + NBUF - K) % NBUF
        gather_wait(b2)
        scatter_start(b2)
        if c + K < N_CHUNKS:
            idx_start(c + K, b2)
        scatter_wait(b)
        compute_sidx(b, wchunk + c)
        idx_wait(b)
        gather_start(b)

    for c in range(N_CHUNKS - K, N_CHUNKS):
        b = c % NBUF
        gather_wait(b)
        scatter_start(b)
    for b in range(NBUF):
        scatter_wait(b)


def kernel(indices, table):
    rtable = _relu_table(table)
    idx_flat = jnp.transpose(indices.astype(jnp.int32), (2, 1, 0)).reshape(-1)
    out = _gather_scatter(idx_flat, rtable)
    return out.reshape(L, B, G * D)
